# Initial kernel scaffold; baseline (speedup 1.0000x reference)
import jax
import jax.numpy as jnp
from jax.experimental import pallas as pl


def kernel(x, norm_weight, gate_weight, gate_bias, mlp1_weight, mlp1_bias, mlp2_weight, mlp2_bias):
    raise NotImplementedError("write your pallas kernel here")


# speedup vs baseline: 11.3969x; 11.3969x over previous
import functools

import jax
import jax.numpy as jnp
from jax.experimental import pallas as pl
from jax.experimental.pallas import tpu as pltpu

D_MODEL = 640
INTERMEDIATE_SIZE = 640
EXPERTS_PER_TOKEN = 4
RMS_NORM_EPS = 1e-05
SWIGLU_LIMIT = 7.0
N_EXPERTS = 16
SEQ_LEN = 32


def _moe_kernel(xt_ref, nw_ref, gw_ref, gb_ref, m1w_ref, m1b_ref, m2w_ref,
                m2b_ref, out_ref, t_s, w_s):
    e = pl.program_id(0)
    T, D, I, E, K = SEQ_LEN, D_MODEL, INTERMEDIATE_SIZE, N_EXPERTS, EXPERTS_PER_TOKEN

    @pl.when(e == 0)
    def _prologue():
        xt = xt_ref[...]
        var = jnp.mean(xt * xt, axis=1, keepdims=True)
        t = xt * jax.lax.rsqrt(var + RMS_NORM_EPS) * nw_ref[...]
        t_s[...] = t
        g = jax.lax.dot_general(t, gw_ref[...], (((1,), (1,)), ((), ())),
                                preferred_element_type=jnp.float32)
        g = g + gb_ref[...]
        lane = jax.lax.broadcasted_iota(jnp.int32, (T, E), 1)
        rank = jnp.zeros((T, E), dtype=jnp.int32)
        for j in range(E):
            gj = g[:, j:j + 1]
            rank = rank + (gj > g).astype(jnp.int32)
            rank = rank + ((gj == g) & (j < lane)).astype(jnp.int32)
        sel = rank < K
        neg = jnp.float32(-jnp.inf)
        gm = jnp.where(sel, g, neg)
        mx = jnp.max(gm, axis=1, keepdims=True)
        ex = jnp.where(sel, jnp.exp(g - mx), 0.0)
        w_s[...] = ex / jnp.sum(ex, axis=1, keepdims=True)

    t = t_s[...]
    h = jnp.dot(t, m1w_ref[0], preferred_element_type=jnp.float32)
    h = h + m1b_ref[0]
    h_glu = jnp.minimum(h[:, :I], SWIGLU_LIMIT)
    h_lin = jnp.clip(h[:, I:], -SWIGLU_LIMIT, SWIGLU_LIMIT)
    act = h_glu * jax.nn.sigmoid(1.702 * h_glu) * (h_lin + 1.0)
    o = jnp.dot(act, m2w_ref[0], preferred_element_type=jnp.float32)
    o = o + m2b_ref[0]
    lane_e = jax.lax.broadcasted_iota(jnp.int32, (T, E), 1)
    wcol = jnp.sum(jnp.where(lane_e == e, w_s[...], 0.0), axis=1,
                   keepdims=True)
    contrib = wcol * o

    @pl.when(e == 0)
    def _init():
        out_ref[...] = xt_ref[...] + contrib

    @pl.when(e != 0)
    def _acc():
        out_ref[...] += contrib


@jax.jit
def kernel(x, norm_weight, gate_weight, gate_bias, mlp1_weight, mlp1_bias,
           mlp2_weight, mlp2_bias):
    T, D, I, E = SEQ_LEN, D_MODEL, INTERMEDIATE_SIZE, N_EXPERTS
    xt = x.reshape(D, T).T
    out = pl.pallas_call(
        _moe_kernel,
        grid=(E,),
        in_specs=[
            pl.BlockSpec((T, D), lambda e: (0, 0)),
            pl.BlockSpec((1, D), lambda e: (0, 0)),
            pl.BlockSpec((E, D), lambda e: (0, 0)),
            pl.BlockSpec((1, E), lambda e: (0, 0)),
            pl.BlockSpec((1, D, 2 * I), lambda e: (e, 0, 0)),
            pl.BlockSpec((1, 1, 2 * I), lambda e: (e, 0, 0)),
            pl.BlockSpec((1, I, D), lambda e: (e, 0, 0)),
            pl.BlockSpec((1, 1, D), lambda e: (e, 0, 0)),
        ],
        out_specs=pl.BlockSpec((T, D), lambda e: (0, 0)),
        out_shape=jax.ShapeDtypeStruct((T, D), jnp.float32),
        scratch_shapes=[
            pltpu.VMEM((T, D), jnp.float32),
            pltpu.VMEM((T, E), jnp.float32),
        ],
        compiler_params=pltpu.CompilerParams(
            dimension_semantics=("arbitrary",),
        ),
    )(xt, norm_weight.reshape(1, D), gate_weight, gate_bias.reshape(1, E),
      mlp1_weight, mlp1_bias.reshape(E, 1, 2 * I), mlp2_weight,
      mlp2_bias.reshape(E, 1, D))
    return out.T.reshape(1, D, 1, T)
